# baseline (speedup 1.0000x reference)
import functools

import jax
import jax.numpy as jnp
from jax.experimental import pallas as pl
from jax.experimental.pallas import tpu as pltpu

TAU = 0.1
VMEM_LIMIT = 100 * 1024 * 1024
BF = jnp.bfloat16


def _tower_kernel(z_ref, w1_ref, w2_ref, w3_ref, o_ref):
    E = w2_ref.shape[0]
    Bt = z_ref.shape[1]
    row = Bt * 24

    z = z_ref[...].reshape(22 * row, 128)

    m1 = 20 * row
    p1 = jnp.concatenate(
        [z[dh * row: dh * row + m1] for dh in (0, 1)], axis=1)
    y1 = jnp.maximum(
        jnp.dot(p1, w1_ref[...], preferred_element_type=jnp.float32),
        0.0).astype(BF)

    m2 = 9 * row
    m3 = 7 * row
    for e in range(E):
        y1e = jnp.pad(y1[:, 32 * e: 32 * e + 32], ((0, 2 * row), (0, 0)))
        v2 = y1e.reshape(11, 2, row, 32)
        pieces = []
        for kh in range(4):
            ph, dh = kh % 2, kh // 2
            s2 = v2[:, ph][dh: dh + 10].reshape(10 * row, 32)
            for kw in range(4):
                pieces.append(s2[kw: kw + m2])
        p2 = jnp.concatenate(pieces, axis=1)
        y2 = jnp.maximum(
            jnp.dot(p2, w2_ref[e], preferred_element_type=jnp.float32),
            0.0).astype(BF)

        y2p = jnp.pad(y2, ((0, 8), (0, 0)))
        p3 = jnp.concatenate(
            [y2p[kh * row + 2 * kw: kh * row + 2 * kw + m3]
             for kh in range(3) for kw in range(3)], axis=1)
        y3 = jnp.maximum(
            jnp.dot(p3, w3_ref[e], preferred_element_type=jnp.float32),
            0.0).astype(BF)

        y3 = y3.reshape(7, Bt, 24, 64)
        y3 = jax.lax.slice(y3, (0, 0, 0, 0), (7, Bt, 16, 64))
        y3 = jnp.transpose(y3, (1, 0, 2, 3))
        o_ref[e] = y3.reshape(Bt, 7 * 16 * 64)


def _tower(z1, w1t, w2, w3, bt):
    E = w2.shape[0]
    B = z1.shape[1]
    return pl.pallas_call(
        _tower_kernel,
        out_shape=jax.ShapeDtypeStruct((E, B, 7168), BF),
        grid=(B // bt,),
        in_specs=[
            pl.BlockSpec((22, bt, 24, 128), lambda i: (0, i, 0, 0)),
            pl.BlockSpec((256, 160), lambda i: (0, 0)),
            pl.BlockSpec((E, 512, 64), lambda i: (0, 0, 0)),
            pl.BlockSpec((E, 576, 64), lambda i: (0, 0, 0)),
        ],
        out_specs=pl.BlockSpec((E, bt, 7168), lambda i: (0, i, 0)),
        compiler_params=pltpu.CompilerParams(
            dimension_semantics=("parallel",),
            vmem_limit_bytes=VMEM_LIMIT),
    )(z1, w1t, w2, w3)


def _fc_kernel(h_ref, w1_ref, b1_ref, w2_ref, b2_ref, o_ref, *, n_ens, tau):
    e = pl.program_id(1)
    hh = jnp.maximum(
        jnp.dot(h_ref[0], w1_ref[0], preferred_element_type=jnp.float32)
        + b1_ref[0], 0.0)
    q = jnp.dot(hh.astype(BF), w2_ref[0],
                preferred_element_type=jnp.float32) + b2_ref[0]
    s = q * (1.0 / tau)
    ex = jnp.exp(s - jnp.max(s, axis=-1, keepdims=True))
    p = ex * pl.reciprocal(jnp.sum(ex, axis=-1, keepdims=True), approx=True)

    @pl.when(e == 0)
    def _():
        o_ref[...] = jnp.zeros_like(o_ref)

    o_ref[...] += p

    @pl.when(e == n_ens - 1)
    def _():
        o_ref[...] *= (1.0 / n_ens)


def _fc_fused(h, w1, b1, w2, b2):
    E, Bb, K = h.shape
    N = w1.shape[2]
    A = w2.shape[2]
    half = Bb // 2
    body = functools.partial(_fc_kernel, n_ens=E, tau=TAU)
    return pl.pallas_call(
        body,
        out_shape=jax.ShapeDtypeStruct((Bb, A), jnp.float32),
        grid=(2, E),
        in_specs=[
            pl.BlockSpec((1, half, K), lambda c, e: (e, c, 0)),
            pl.BlockSpec((1, K, N), lambda c, e: (e, 0, 0)),
            pl.BlockSpec((1, 1, N), lambda c, e: (e, 0, 0)),
            pl.BlockSpec((1, N, A), lambda c, e: (e, 0, 0)),
            pl.BlockSpec((1, 1, A), lambda c, e: (e, 0, 0)),
        ],
        out_specs=pl.BlockSpec((half, A), lambda c, e: (c, 0)),
        compiler_params=pltpu.CompilerParams(
            dimension_semantics=("parallel", "arbitrary"),
            vmem_limit_bytes=VMEM_LIMIT),
    )(h, w1, b1, w2, b2)


def kernel(x, w1_all, w2, w3, fc1_w, fc1_b, fc2_w, fc2_b):
    E = w2.shape[0]
    B = x.shape[0]
    bt = 16 if B % 16 == 0 else 8

    xr = x.astype(BF).reshape(B, 4, 21, 4, 21, 4)
    z1 = xr.transpose(2, 0, 4, 1, 3, 5).reshape(21, B, 21, 64)
    z1p = jnp.pad(z1, ((0, 0), (0, 0), (0, 1), (0, 0)))
    z1d = jnp.concatenate([z1p[:, :, 0:21], z1p[:, :, 1:22]], axis=3)
    z1d = jnp.pad(z1d, ((0, 1), (0, 0), (0, 3), (0, 0)))

    w1t = (w1_all.reshape(2, 4, 2, 4, 4, 160)
           .transpose(0, 2, 4, 1, 3, 5).reshape(256, 160).astype(BF))

    h = _tower(z1d, w1t, w2.astype(BF), w3.astype(BF), bt)

    w1v = fc1_w.reshape(E, 7, 7, 64, 512).astype(BF)
    w1p = jnp.stack([w1v, jnp.zeros_like(w1v)], axis=3)
    w1p = jnp.pad(w1p.reshape(E, 7, 14, 64, 512),
                  ((0, 0), (0, 0), (0, 2), (0, 0), (0, 0)))
    w1p = w1p.reshape(E, 7168, 512)

    return _fc_fused(h, w1p, fc1_b, fc2_w.astype(BF), fc2_b)


# speedup vs baseline: 1.0074x; 1.0074x over previous
import functools

import jax
import jax.numpy as jnp
from jax.experimental import pallas as pl
from jax.experimental.pallas import tpu as pltpu

TAU = 0.1
VMEM_LIMIT = 100 * 1024 * 1024
BF = jnp.bfloat16


def _tower_kernel(z_ref, w1_ref, w2_ref, w3_ref, o_ref):
    E = w2_ref.shape[0]
    Bt = z_ref.shape[1]
    row = Bt * 24

    z = z_ref[...].reshape(22 * row, 128)

    m1 = 20 * row
    p1 = jnp.concatenate(
        [z[dh * row: dh * row + m1] for dh in (0, 1)], axis=1)
    y1 = jnp.maximum(
        jnp.dot(p1, w1_ref[...], preferred_element_type=jnp.float32),
        0.0).astype(BF)

    m2 = 9 * row
    m3 = 7 * row
    for e in range(E):
        y1e = jnp.pad(y1[:, 32 * e: 32 * e + 32], ((0, 2 * row), (0, 0)))
        v2 = y1e.reshape(11, 2, row, 32)
        pieces = []
        for kh in range(4):
            ph, dh = kh % 2, kh // 2
            s2 = v2[:, ph][dh: dh + 10].reshape(10 * row, 32)
            for kw in range(4):
                pieces.append(s2[kw: kw + m2])
        p2 = jnp.concatenate(pieces, axis=1)
        y2 = jnp.maximum(
            jnp.dot(p2, w2_ref[e], preferred_element_type=jnp.float32),
            0.0).astype(BF)

        y2p = jnp.pad(y2, ((0, 8), (0, 0)))
        p3 = jnp.concatenate(
            [y2p[kh * row + 2 * kw: kh * row + 2 * kw + m3]
             for kh in range(3) for kw in range(3)], axis=1)
        y3 = jnp.maximum(
            jnp.dot(p3, w3_ref[e], preferred_element_type=jnp.float32),
            0.0).astype(BF)

        y3 = y3.reshape(7, Bt, 24, 64)
        y3 = jax.lax.slice(y3, (0, 0, 0, 0), (7, Bt, 16, 64))
        y3 = jnp.transpose(y3, (1, 0, 2, 3))
        o_ref[e] = y3.reshape(Bt, 7 * 16 * 64)


def _tower(z1, w1t, w2, w3, bt):
    E = w2.shape[0]
    B = z1.shape[1]
    return pl.pallas_call(
        _tower_kernel,
        out_shape=jax.ShapeDtypeStruct((E, B, 7168), BF),
        grid=(B // bt,),
        in_specs=[
            pl.BlockSpec((22, bt, 24, 128), lambda i: (0, i, 0, 0)),
            pl.BlockSpec((256, 160), lambda i: (0, 0)),
            pl.BlockSpec((E, 512, 64), lambda i: (0, 0, 0)),
            pl.BlockSpec((E, 576, 64), lambda i: (0, 0, 0)),
        ],
        out_specs=pl.BlockSpec((E, bt, 7168), lambda i: (0, i, 0)),
        compiler_params=pltpu.CompilerParams(
            dimension_semantics=("parallel",),
            vmem_limit_bytes=VMEM_LIMIT),
    )(z1, w1t, w2, w3)


def _fc_kernel(h_ref, w1_ref, b1_ref, w2_ref, b2_ref, o_ref, *, n_ens, tau):
    e = pl.program_id(1)
    hh = jnp.maximum(
        jnp.dot(h_ref[0], w1_ref[0], preferred_element_type=jnp.float32)
        + b1_ref[0], 0.0)
    q = jnp.dot(hh.astype(BF), w2_ref[0],
                preferred_element_type=jnp.float32) + b2_ref[0]
    s = q * (1.0 / tau)
    ex = jnp.exp(s - jnp.max(s, axis=-1, keepdims=True))
    p = ex * pl.reciprocal(jnp.sum(ex, axis=-1, keepdims=True), approx=True)

    @pl.when(e == 0)
    def _():
        o_ref[...] = jnp.zeros_like(o_ref)

    o_ref[...] += p

    @pl.when(e == n_ens - 1)
    def _():
        o_ref[...] *= (1.0 / n_ens)


def _fc_fused(h, w1, b1, w2, b2):
    E, Bb, K = h.shape
    N = w1.shape[2]
    A = w2.shape[2]
    half = Bb // 2
    body = functools.partial(_fc_kernel, n_ens=E, tau=TAU)
    return pl.pallas_call(
        body,
        out_shape=jax.ShapeDtypeStruct((Bb, A), jnp.float32),
        grid=(2, E),
        in_specs=[
            pl.BlockSpec((1, half, K), lambda c, e: (e, c, 0)),
            pl.BlockSpec((1, K, N), lambda c, e: (e, 0, 0)),
            pl.BlockSpec((1, 1, N), lambda c, e: (e, 0, 0)),
            pl.BlockSpec((1, N, A), lambda c, e: (e, 0, 0)),
            pl.BlockSpec((1, 1, A), lambda c, e: (e, 0, 0)),
        ],
        out_specs=pl.BlockSpec((half, A), lambda c, e: (c, 0)),
        compiler_params=pltpu.CompilerParams(
            dimension_semantics=("parallel", "arbitrary"),
            vmem_limit_bytes=VMEM_LIMIT),
    )(h, w1, b1, w2, b2)


def kernel(x, w1_all, w2, w3, fc1_w, fc1_b, fc2_w, fc2_b):
    E = w2.shape[0]
    B = x.shape[0]
    bt = 32 if B % 32 == 0 else 8

    xr = x.astype(BF).reshape(B, 4, 21, 4, 21, 4)
    z1 = xr.transpose(2, 0, 4, 1, 3, 5).reshape(21, B, 21, 64)
    z1p = jnp.pad(z1, ((0, 0), (0, 0), (0, 1), (0, 0)))
    z1d = jnp.concatenate([z1p[:, :, 0:21], z1p[:, :, 1:22]], axis=3)
    z1d = jnp.pad(z1d, ((0, 1), (0, 0), (0, 3), (0, 0)))

    w1t = (w1_all.reshape(2, 4, 2, 4, 4, 160)
           .transpose(0, 2, 4, 1, 3, 5).reshape(256, 160).astype(BF))

    h = _tower(z1d, w1t, w2.astype(BF), w3.astype(BF), bt)

    w1v = fc1_w.reshape(E, 7, 7, 64, 512).astype(BF)
    w1p = jnp.stack([w1v, jnp.zeros_like(w1v)], axis=3)
    w1p = jnp.pad(w1p.reshape(E, 7, 14, 64, 512),
                  ((0, 0), (0, 0), (0, 2), (0, 0), (0, 0)))
    w1p = w1p.reshape(E, 7168, 512)

    return _fc_fused(h, w1p, fc1_b, fc2_w.astype(BF), fc2_b)
